# baseline (speedup 1.0000x reference)
import jax
import jax.numpy as jnp
from jax.experimental import pallas as pl
from jax.experimental.pallas import tpu as pltpu

KK = 25
PAD = 12
WOFF = 16
RT = 16
CT = 16
NB = 256
SH = 40
SW = 48
KDIM = SH * SW
HCH = 8


def _round_up(x, m):
    return ((x + m - 1) // m) * m


def _banded_a(weight):
    w2 = weight.reshape(KK, KK).astype(jnp.float32)
    dxs = jnp.arange(KK)
    cs = jnp.arange(CT)
    wls = jnp.arange(SW)
    xm = (wls[None, None, :] == cs[None, :, None] + dxs[:, None, None] + 4)
    dys = jnp.arange(KK)
    rs = jnp.arange(RT)
    hls = jnp.arange(SH)
    ym = (hls[None, None, :] == rs[None, :, None] + dys[:, None, None])
    t1 = jnp.einsum('yx,xcw->ycw', w2, xm.astype(jnp.float32))
    a4 = jnp.einsum('ycw,yrh->rchw', t1, ym.astype(jnp.float32))
    return a4.reshape(RT * CT, KDIM).astype(jnp.bfloat16)


def _make_kernel(H, W, n_bands):
    n_chunks = H // HCH

    def _in_copy(x_ref, s1_ref, sem_in, nb0, h):
        return pltpu.make_async_copy(
            x_ref.at[pl.ds(nb0, NB), 0, h, :], s1_ref.at[h],
            sem_in.at[h // HCH, h % HCH])

    def _out_copy(o_ref, s2_ref, sem_out, nb0, band, j):
        return pltpu.make_async_copy(
            s2_ref.at[band % 2, j], o_ref.at[pl.ds(nb0, NB), 0, band * RT + j, :],
            sem_out.at[band % 2, j])

    def _conv_sig_kernel(a_ref, b_ref, x_ref, o_ref,
                         xp_ref, s1_ref, band_ref, s2_ref, sem_in, sem_out):
        k = pl.program_id(1)
        b = pl.program_id(0)
        nb0 = b * NB
        Hp, Wp, _ = xp_ref.shape
        last = n_bands - 1

        @pl.when(k == 0)
        def _start():
            xp_ref[:PAD, :, :] = jnp.zeros((PAD, Wp, NB), jnp.bfloat16)
            xp_ref[PAD + H:, :, :] = jnp.zeros(
                (Hp - PAD - H, Wp, NB), jnp.bfloat16)
            xp_ref[PAD:PAD + H, :WOFF, :] = jnp.zeros((H, WOFF, NB),
                                                      jnp.bfloat16)
            xp_ref[PAD:PAD + H, WOFF + W:, :] = jnp.zeros(
                (H, Wp - WOFF - W, NB), jnp.bfloat16)
            for h in range(H):
                _in_copy(x_ref, s1_ref, sem_in, nb0, h).start()
            for c in range(4):
                for h in range(c * HCH, (c + 1) * HCH):
                    _in_copy(x_ref, s1_ref, sem_in, nb0, h).wait()
                h0 = c * HCH
                for ns in range(0, NB, 128):
                    t = jnp.transpose(
                        s1_ref[h0:h0 + HCH, ns:ns + 128, :], (0, 2, 1))
                    xp_ref[PAD + h0:PAD + h0 + HCH, WOFF:WOFF + W,
                           ns:ns + 128] = t.astype(jnp.bfloat16)

        @pl.when(k <= (n_chunks - 6) // 2)
        def _wait_chunks():
            for dc in range(2):
                for dh in range(HCH):
                    h = (2 * k + 4 + dc) * HCH + dh
                    pltpu.make_async_copy(
                        x_ref.at[pl.ds(nb0, NB), 0, h, :],
                        s1_ref.at[h],
                        sem_in.at[2 * k + 4 + dc, dh]).wait()

        @pl.when(k >= 3)
        def _reclaim():
            for j in range(RT):
                _out_copy(o_ref, s2_ref, sem_out, nb0, k - 3, j).wait()

        def _step(par):
            bias = b_ref[0]
            a = a_ref[...]
            hb = k * RT
            h0 = jnp.minimum((2 * k + 4) * HCH, H - 2 * HCH)
            n_tiles = W // CT
            tp = [(j, ns) for j in range(RT) for ns in (0, 128)]
            for wt in range(n_tiles):
                wb = wt * CT
                slab = xp_ref[pl.ds(hb, SH), wb:wb + SW, :].reshape(KDIM, NB)
                acc = jnp.dot(a, slab, preferred_element_type=jnp.float32)
                band_ref[par, :, wb:wb + CT, :] = (
                    jax.nn.sigmoid(acc + bias).reshape(RT, CT, NB))
                for j, ns in tp[wt * 4:(wt + 1) * 4]:
                    piece = band_ref[1 - par, j, :, ns:ns + 128]
                    s2_ref[1 - par, j, pl.ds(ns, 128), :] = piece.T
                if wt in (2, 5):
                    ns = 0 if wt == 2 else 128
                    t = jnp.transpose(
                        s1_ref[pl.ds(h0, 2 * HCH), ns:ns + 128, :], (0, 2, 1))
                    xp_ref[pl.ds(PAD + h0, 2 * HCH), WOFF:WOFF + W,
                           ns:ns + 128] = t.astype(jnp.bfloat16)

        @pl.when(k % 2 == 0)
        def _even():
            _step(0)

        @pl.when(k % 2 == 1)
        def _odd():
            _step(1)

        @pl.when(k >= 1)
        def _ship():
            for j in range(RT):
                _out_copy(o_ref, s2_ref, sem_out, nb0, k - 1, j).start()

        @pl.when(k == last)
        def _drain():
            for j in range(RT):
                _out_copy(o_ref, s2_ref, sem_out, nb0, last - 2, j).wait()
            for j in range(RT):
                for ns in range(0, NB, 128):
                    piece = band_ref[last % 2, j, :, ns:ns + 128]
                    s2_ref[last % 2, j, pl.ds(ns, 128), :] = piece.T
            for j in range(RT):
                _out_copy(o_ref, s2_ref, sem_out, nb0, last, j).start()
            for kk in (last - 1, last):
                for j in range(RT):
                    _out_copy(o_ref, s2_ref, sem_out, nb0, kk, j).wait()

    return _conv_sig_kernel


def _forward(x_nchw, weight, bias):
    N, C, H, W = x_nchw.shape
    assert C == 1
    Hp = _round_up(PAD + H + PAD, 8)
    Wp = _round_up(WOFF + W + PAD, 8)
    n_bands = H // RT

    a_mat = _banded_a(weight)

    Np = _round_up(N, NB)
    x = x_nchw
    if Np != N:
        x = jnp.pad(x, ((0, Np - N), (0, 0), (0, 0), (0, 0)))

    out = pl.pallas_call(
        _make_kernel(H, W, n_bands),
        out_shape=jax.ShapeDtypeStruct((Np, 1, H, W), x_nchw.dtype),
        grid=(Np // NB, n_bands),
        in_specs=[
            pl.BlockSpec((RT * CT, KDIM), lambda b, h: (0, 0)),
            pl.BlockSpec(memory_space=pltpu.MemorySpace.SMEM),
            pl.BlockSpec(memory_space=pltpu.MemorySpace.HBM),
        ],
        out_specs=pl.BlockSpec(memory_space=pltpu.MemorySpace.HBM),
        scratch_shapes=[
            pltpu.VMEM((Hp, Wp, NB), jnp.bfloat16),
            pltpu.VMEM((H, NB, W), jnp.float32),
            pltpu.VMEM((2, RT, W, NB), jnp.float32),
            pltpu.VMEM((2, RT, NB, W), jnp.float32),
            pltpu.SemaphoreType.DMA((H // HCH, HCH)),
            pltpu.SemaphoreType.DMA((2, RT)),
        ],
        compiler_params=pltpu.CompilerParams(
            dimension_semantics=("parallel", "arbitrary")),
    )(a_mat, bias.astype(jnp.float32), x)

    if Np != N:
        out = out[:N]
    return out


def kernel(x_nchw, weight, bias):
    return _forward(x_nchw, weight, bias)


# speedup vs baseline: 1.1312x; 1.1312x over previous
import jax
import jax.numpy as jnp
from jax.experimental import pallas as pl
from jax.experimental.pallas import tpu as pltpu

KK = 25
PAD = 12
WOFF = 16
RT = 16
CT = 16
NB = 256
SH = 40
SW = 48
KDIM = SH * SW
HCH = 8


def _round_up(x, m):
    return ((x + m - 1) // m) * m


def _banded_a(weight):
    w2 = weight.reshape(KK, KK).astype(jnp.float32)
    dxs = jnp.arange(KK)
    cs = jnp.arange(CT)
    wls = jnp.arange(SW)
    xm = (wls[None, None, :] == cs[None, :, None] + dxs[:, None, None] + 4)
    dys = jnp.arange(KK)
    rs = jnp.arange(RT)
    hls = jnp.arange(SH)
    ym = (hls[None, None, :] == rs[None, :, None] + dys[:, None, None])
    t1 = jnp.einsum('yx,xcw->ycw', w2, xm.astype(jnp.float32))
    a4 = jnp.einsum('ycw,yrh->rchw', t1, ym.astype(jnp.float32))
    return a4.reshape(RT * CT, KDIM).astype(jnp.bfloat16)


def _make_kernel(H, W, n_bands):
    n_chunks = H // HCH

    def _in_copy(x_ref, s1_ref, sem_in, nb0, h):
        return pltpu.make_async_copy(
            x_ref.at[pl.ds(nb0, NB), 0, h, :], s1_ref.at[h],
            sem_in.at[h // HCH, h % HCH])

    def _out_copy(o_ref, s2_ref, sem_out, nb0, band, j):
        return pltpu.make_async_copy(
            s2_ref.at[band % 2, j], o_ref.at[pl.ds(nb0, NB), 0, band * RT + j, :],
            sem_out.at[band % 2, j])

    def _conv_sig_kernel(a_ref, b_ref, x_ref, o_ref,
                         xp_ref, s1_ref, band_ref, s2_ref, sem_in, sem_out):
        k = pl.program_id(1)
        b = pl.program_id(0)
        nb0 = b * NB
        Hp, Wp, _ = xp_ref.shape
        last = n_bands - 1

        @pl.when(k == 0)
        def _start():
            xp_ref[:PAD, :, :] = jnp.zeros((PAD, Wp, NB), jnp.bfloat16)
            xp_ref[PAD + H:, :, :] = jnp.zeros(
                (Hp - PAD - H, Wp, NB), jnp.bfloat16)
            xp_ref[PAD:PAD + H, :WOFF, :] = jnp.zeros((H, WOFF, NB),
                                                      jnp.bfloat16)
            xp_ref[PAD:PAD + H, WOFF + W:, :] = jnp.zeros(
                (H, Wp - WOFF - W, NB), jnp.bfloat16)
            for h in range(H):
                _in_copy(x_ref, s1_ref, sem_in, nb0, h).start()
            for c in range(4):
                for h in range(c * HCH, (c + 1) * HCH):
                    _in_copy(x_ref, s1_ref, sem_in, nb0, h).wait()
                h0 = c * HCH
                for ns in range(0, NB, 128):
                    t = jnp.transpose(
                        s1_ref[h0:h0 + HCH, ns:ns + 128, :], (0, 2, 1))
                    xp_ref[PAD + h0:PAD + h0 + HCH, WOFF:WOFF + W,
                           ns:ns + 128] = t.astype(jnp.bfloat16)

        @pl.when(k <= (n_chunks - 6) // 2)
        def _wait_chunks():
            for dc in range(2):
                for dh in range(HCH):
                    h = (2 * k + 4 + dc) * HCH + dh
                    pltpu.make_async_copy(
                        x_ref.at[pl.ds(nb0, NB), 0, h, :],
                        s1_ref.at[h],
                        sem_in.at[2 * k + 4 + dc, dh]).wait()

        @pl.when(k >= 3)
        def _reclaim():
            for j in range(RT):
                _out_copy(o_ref, s2_ref, sem_out, nb0, k - 3, j).wait()

        def _step(par):
            for j in range(RT):
                for ns in range(0, NB, 128):
                    piece = band_ref[1 - par, j, :, ns:ns + 128]
                    s2_ref[1 - par, j, pl.ds(ns, 128), :] = piece.T

            bias = b_ref[0]
            a = a_ref[...]
            hb = k * RT
            for wt in range(W // CT):
                wb = wt * CT
                slab = xp_ref[pl.ds(hb, SH), wb:wb + SW, :].reshape(KDIM, NB)
                acc = jnp.dot(a, slab, preferred_element_type=jnp.float32)
                band_ref[par, :, wb:wb + CT, :] = (
                    jax.nn.sigmoid(acc + bias).reshape(RT, CT, NB))

            h0 = jnp.minimum((2 * k + 4) * HCH, H - 2 * HCH)
            for ns in range(0, NB, 128):
                t = jnp.transpose(
                    s1_ref[pl.ds(h0, 2 * HCH), ns:ns + 128, :], (0, 2, 1))
                xp_ref[pl.ds(PAD + h0, 2 * HCH), WOFF:WOFF + W,
                       ns:ns + 128] = t.astype(jnp.bfloat16)

        @pl.when(k % 2 == 0)
        def _even():
            _step(0)

        @pl.when(k % 2 == 1)
        def _odd():
            _step(1)

        @pl.when(k >= 1)
        def _ship():
            for j in range(RT):
                _out_copy(o_ref, s2_ref, sem_out, nb0, k - 1, j).start()

        @pl.when(k == last)
        def _drain():
            for j in range(RT):
                _out_copy(o_ref, s2_ref, sem_out, nb0, last - 2, j).wait()
            for j in range(RT):
                for ns in range(0, NB, 128):
                    piece = band_ref[last % 2, j, :, ns:ns + 128]
                    s2_ref[last % 2, j, pl.ds(ns, 128), :] = piece.T
            for j in range(RT):
                _out_copy(o_ref, s2_ref, sem_out, nb0, last, j).start()
            for kk in (last - 1, last):
                for j in range(RT):
                    _out_copy(o_ref, s2_ref, sem_out, nb0, kk, j).wait()

    return _conv_sig_kernel


def _forward(x_nchw, weight, bias):
    N, C, H, W = x_nchw.shape
    assert C == 1
    Hp = _round_up(PAD + H + PAD, 8)
    Wp = _round_up(WOFF + W + PAD, 8)
    n_bands = H // RT

    a_mat = _banded_a(weight)

    Np = _round_up(N, NB)
    x = x_nchw
    if Np != N:
        x = jnp.pad(x, ((0, Np - N), (0, 0), (0, 0), (0, 0)))

    out = pl.pallas_call(
        _make_kernel(H, W, n_bands),
        out_shape=jax.ShapeDtypeStruct((Np, 1, H, W), x_nchw.dtype),
        grid=(Np // NB, n_bands),
        in_specs=[
            pl.BlockSpec((RT * CT, KDIM), lambda b, h: (0, 0)),
            pl.BlockSpec(memory_space=pltpu.MemorySpace.SMEM),
            pl.BlockSpec(memory_space=pltpu.MemorySpace.HBM),
        ],
        out_specs=pl.BlockSpec(memory_space=pltpu.MemorySpace.HBM),
        scratch_shapes=[
            pltpu.VMEM((Hp, Wp, NB), jnp.bfloat16),
            pltpu.VMEM((H, NB, W), jnp.float32),
            pltpu.VMEM((2, RT, W, NB), jnp.float32),
            pltpu.VMEM((2, RT, NB, W), jnp.float32),
            pltpu.SemaphoreType.DMA((H // HCH, HCH)),
            pltpu.SemaphoreType.DMA((2, RT)),
        ],
        compiler_params=pltpu.CompilerParams(
            dimension_semantics=("parallel", "arbitrary")),
    )(a_mat, bias.astype(jnp.float32), x)

    if Np != N:
        out = out[:N]
    return out


def kernel(x_nchw, weight, bias):
    return _forward(x_nchw, weight, bias)


# speedup vs baseline: 1.1355x; 1.0038x over previous
import jax
import jax.numpy as jnp
from jax.experimental import pallas as pl
from jax.experimental.pallas import tpu as pltpu

KK = 25
PAD = 12
WOFF = 16
RT = 16
CT = 16
NB = 256
SH = 40
SW = 48
KDIM = SH * SW
HCH = 8


def _round_up(x, m):
    return ((x + m - 1) // m) * m


def _banded_a(weight):
    w2 = weight.reshape(KK, KK).astype(jnp.float32)
    dxs = jnp.arange(KK)
    cs = jnp.arange(CT)
    wls = jnp.arange(SW)
    xm = (wls[None, None, :] == cs[None, :, None] + dxs[:, None, None] + 4)
    dys = jnp.arange(KK)
    rs = jnp.arange(RT)
    hls = jnp.arange(SH)
    ym = (hls[None, None, :] == rs[None, :, None] + dys[:, None, None])
    t1 = jnp.einsum('yx,xcw->ycw', w2, xm.astype(jnp.float32))
    a4 = jnp.einsum('ycw,yrh->rchw', t1, ym.astype(jnp.float32))
    return a4.reshape(RT * CT, KDIM).astype(jnp.bfloat16)


def _make_kernel(H, W, n_bands):
    n_chunks = H // HCH

    def _in_copy(x_ref, s1_ref, sem_in, nb0, h):
        return pltpu.make_async_copy(
            x_ref.at[pl.ds(nb0, NB), 0, h, :], s1_ref.at[h],
            sem_in.at[h // HCH, h % HCH])

    def _out_copy(o_ref, s2_ref, sem_out, nb0, band, j):
        return pltpu.make_async_copy(
            s2_ref.at[band % 2, j], o_ref.at[pl.ds(nb0, NB), 0, band * RT + j, :],
            sem_out.at[band % 2, j])

    def _conv_sig_kernel(a_ref, b_ref, x_ref, o_ref,
                         xp_ref, s1_ref, band_ref, s2_ref, sem_in, sem_out):
        k = pl.program_id(1)
        b = pl.program_id(0)
        nb0 = b * NB
        Hp, Wp, _ = xp_ref.shape
        last = n_bands - 1

        @pl.when(k == 0)
        def _start():
            xp_ref[:PAD, :, :] = jnp.zeros((PAD, Wp, NB), jnp.bfloat16)
            xp_ref[PAD + H:, :, :] = jnp.zeros(
                (Hp - PAD - H, Wp, NB), jnp.bfloat16)
            xp_ref[PAD:PAD + H, :WOFF, :] = jnp.zeros((H, WOFF, NB),
                                                      jnp.bfloat16)
            xp_ref[PAD:PAD + H, WOFF + W:, :] = jnp.zeros(
                (H, Wp - WOFF - W, NB), jnp.bfloat16)
            for h in range(H):
                _in_copy(x_ref, s1_ref, sem_in, nb0, h).start()
            for c in range(4):
                for h in range(c * HCH, (c + 1) * HCH):
                    _in_copy(x_ref, s1_ref, sem_in, nb0, h).wait()
                h0 = c * HCH
                for ns in range(0, NB, 128):
                    t = jnp.transpose(
                        s1_ref[h0:h0 + HCH, ns:ns + 128, :], (0, 2, 1))
                    xp_ref[PAD + h0:PAD + h0 + HCH, WOFF:WOFF + W,
                           ns:ns + 128] = t.astype(jnp.bfloat16)

        @pl.when(k <= (n_chunks - 6) // 2)
        def _wait_chunks():
            for dc in range(2):
                for dh in range(HCH):
                    h = (2 * k + 4 + dc) * HCH + dh
                    pltpu.make_async_copy(
                        x_ref.at[pl.ds(nb0, NB), 0, h, :],
                        s1_ref.at[h],
                        sem_in.at[2 * k + 4 + dc, dh]).wait()

        @pl.when(k >= 3)
        def _reclaim():
            for j in range(RT):
                _out_copy(o_ref, s2_ref, sem_out, nb0, k - 3, j).wait()

        def _step(par):
            for j in range(RT):
                for ns in range(0, NB, 128):
                    piece = band_ref[1 - par, j, :, ns:ns + 128]
                    s2_ref[1 - par, j, pl.ds(ns, 128), :] = piece.T

            bias = b_ref[0]
            a = a_ref[...]
            hb = k * RT
            for wt in range(W // CT):
                wb = wt * CT
                slab = xp_ref[pl.ds(hb, SH), wb:wb + SW, :].reshape(KDIM, NB)
                acc = jnp.dot(a, slab, preferred_element_type=jnp.float32)
                sig = 0.5 * jnp.tanh(0.5 * acc + 0.5 * bias) + 0.5
                band_ref[par, :, wb:wb + CT, :] = sig.reshape(RT, CT, NB)

            h0 = jnp.minimum((2 * k + 4) * HCH, H - 2 * HCH)
            for ns in range(0, NB, 128):
                t = jnp.transpose(
                    s1_ref[pl.ds(h0, 2 * HCH), ns:ns + 128, :], (0, 2, 1))
                xp_ref[pl.ds(PAD + h0, 2 * HCH), WOFF:WOFF + W,
                       ns:ns + 128] = t.astype(jnp.bfloat16)

        @pl.when(k % 2 == 0)
        def _even():
            _step(0)

        @pl.when(k % 2 == 1)
        def _odd():
            _step(1)

        @pl.when(k >= 1)
        def _ship():
            for j in range(RT):
                _out_copy(o_ref, s2_ref, sem_out, nb0, k - 1, j).start()

        @pl.when(k == last)
        def _drain():
            for j in range(RT):
                _out_copy(o_ref, s2_ref, sem_out, nb0, last - 2, j).wait()
            for j in range(RT):
                for ns in range(0, NB, 128):
                    piece = band_ref[last % 2, j, :, ns:ns + 128]
                    s2_ref[last % 2, j, pl.ds(ns, 128), :] = piece.T
            for j in range(RT):
                _out_copy(o_ref, s2_ref, sem_out, nb0, last, j).start()
            for kk in (last - 1, last):
                for j in range(RT):
                    _out_copy(o_ref, s2_ref, sem_out, nb0, kk, j).wait()

    return _conv_sig_kernel


def _forward(x_nchw, weight, bias):
    N, C, H, W = x_nchw.shape
    assert C == 1
    Hp = _round_up(PAD + H + PAD, 8)
    Wp = _round_up(WOFF + W + PAD, 8)
    n_bands = H // RT

    a_mat = _banded_a(weight)

    Np = _round_up(N, NB)
    x = x_nchw
    if Np != N:
        x = jnp.pad(x, ((0, Np - N), (0, 0), (0, 0), (0, 0)))

    out = pl.pallas_call(
        _make_kernel(H, W, n_bands),
        out_shape=jax.ShapeDtypeStruct((Np, 1, H, W), x_nchw.dtype),
        grid=(Np // NB, n_bands),
        in_specs=[
            pl.BlockSpec((RT * CT, KDIM), lambda b, h: (0, 0)),
            pl.BlockSpec(memory_space=pltpu.MemorySpace.SMEM),
            pl.BlockSpec(memory_space=pltpu.MemorySpace.HBM),
        ],
        out_specs=pl.BlockSpec(memory_space=pltpu.MemorySpace.HBM),
        scratch_shapes=[
            pltpu.VMEM((Hp, Wp, NB), jnp.bfloat16),
            pltpu.VMEM((H, NB, W), jnp.float32),
            pltpu.VMEM((2, RT, W, NB), jnp.float32),
            pltpu.VMEM((2, RT, NB, W), jnp.float32),
            pltpu.SemaphoreType.DMA((H // HCH, HCH)),
            pltpu.SemaphoreType.DMA((2, RT)),
        ],
        compiler_params=pltpu.CompilerParams(
            dimension_semantics=("parallel", "arbitrary")),
    )(a_mat, bias.astype(jnp.float32), x)

    if Np != N:
        out = out[:N]
    return out


def kernel(x_nchw, weight, bias):
    return _forward(x_nchw, weight, bias)


# speedup vs baseline: 1.1359x; 1.0004x over previous
import jax
import jax.numpy as jnp
from jax.experimental import pallas as pl
from jax.experimental.pallas import tpu as pltpu

KK = 25
PAD = 12
WOFF = 16
RT = 16
CT = 16
NB = 256
SH = 40
SW = 48
KDIM = SH * SW
HCH = 8


def _round_up(x, m):
    return ((x + m - 1) // m) * m


def _banded_a(weight):
    w2 = weight.reshape(KK, KK).astype(jnp.float32)
    dxs = jnp.arange(KK)
    cs = jnp.arange(CT)
    wls = jnp.arange(SW)
    xm = (wls[None, None, :] == cs[None, :, None] + dxs[:, None, None] + 4)
    dys = jnp.arange(KK)
    rs = jnp.arange(RT)
    hls = jnp.arange(SH)
    ym = (hls[None, None, :] == rs[None, :, None] + dys[:, None, None])
    t1 = jnp.einsum('yx,xcw->ycw', w2, xm.astype(jnp.float32))
    a4 = jnp.einsum('ycw,yrh->rchw', t1, ym.astype(jnp.float32))
    return a4.reshape(RT * CT, KDIM).astype(jnp.bfloat16)


def _make_kernel(H, W, n_bands):
    n_chunks = H // HCH

    def _in_copy(x_ref, s1_ref, sem_in, nb0, h):
        return pltpu.make_async_copy(
            x_ref.at[pl.ds(nb0, NB), 0, h, :], s1_ref.at[h],
            sem_in.at[h // HCH, h % HCH])

    def _out_copy(o_ref, s2_ref, sem_out, nb0, band, j):
        return pltpu.make_async_copy(
            s2_ref.at[band % 2, j], o_ref.at[pl.ds(nb0, NB), 0, band * RT + j, :],
            sem_out.at[band % 2, j])

    def _conv_sig_kernel(a_ref, b_ref, x_ref, o_ref,
                         xp_ref, s1_ref, band_ref, s2_ref, sem_in, sem_out):
        k = pl.program_id(1)
        b = pl.program_id(0)
        nb0 = b * NB
        Hp, Wp, _ = xp_ref.shape
        last = n_bands - 1

        @pl.when(k == 0)
        def _start():
            xp_ref[:PAD, :, :] = jnp.zeros((PAD, Wp, NB), jnp.bfloat16)
            xp_ref[PAD + H:, :, :] = jnp.zeros(
                (Hp - PAD - H, Wp, NB), jnp.bfloat16)
            xp_ref[PAD:PAD + H, :WOFF, :] = jnp.zeros((H, WOFF, NB),
                                                      jnp.bfloat16)
            xp_ref[PAD:PAD + H, WOFF + W:, :] = jnp.zeros(
                (H, Wp - WOFF - W, NB), jnp.bfloat16)
            for h in range(H):
                _in_copy(x_ref, s1_ref, sem_in, nb0, h).start()
            for c in range(4):
                for h in range(c * HCH, (c + 1) * HCH):
                    _in_copy(x_ref, s1_ref, sem_in, nb0, h).wait()
                h0 = c * HCH
                for ns in range(0, NB, 128):
                    t = jnp.transpose(
                        s1_ref[h0:h0 + HCH, ns:ns + 128, :], (0, 2, 1))
                    xp_ref[PAD + h0:PAD + h0 + HCH, WOFF:WOFF + W,
                           ns:ns + 128] = t.astype(jnp.bfloat16)

        @pl.when(k <= (n_chunks - 6) // 2)
        def _wait_chunks():
            for dc in range(2):
                for dh in range(HCH):
                    h = (2 * k + 4 + dc) * HCH + dh
                    pltpu.make_async_copy(
                        x_ref.at[pl.ds(nb0, NB), 0, h, :],
                        s1_ref.at[h],
                        sem_in.at[2 * k + 4 + dc, dh]).wait()

        @pl.when(k >= 3)
        def _reclaim():
            for j in range(RT):
                _out_copy(o_ref, s2_ref, sem_out, nb0, k - 3, j).wait()

        def _step(par):
            bias = b_ref[0]
            a = a_ref[...]
            hb = k * RT
            for wt in range(W // CT):
                wb = wt * CT
                slab = xp_ref[pl.ds(hb, SH), wb:wb + SW, :].reshape(KDIM, NB)
                acc = jnp.dot(a, slab, preferred_element_type=jnp.float32)
                sig = 0.5 * jnp.tanh(0.5 * acc + 0.5 * bias) + 0.5
                band_ref[par, :, wb:wb + CT, :] = sig.reshape(RT, CT, NB)

            for j in range(RT):
                for ns in range(0, NB, 128):
                    piece = band_ref[1 - par, j, :, ns:ns + 128]
                    s2_ref[1 - par, j, pl.ds(ns, 128), :] = piece.T

            h0 = jnp.minimum((2 * k + 4) * HCH, H - 2 * HCH)
            for ns in range(0, NB, 128):
                t = jnp.transpose(
                    s1_ref[pl.ds(h0, 2 * HCH), ns:ns + 128, :], (0, 2, 1))
                xp_ref[pl.ds(PAD + h0, 2 * HCH), WOFF:WOFF + W,
                       ns:ns + 128] = t.astype(jnp.bfloat16)

        @pl.when(k % 2 == 0)
        def _even():
            _step(0)

        @pl.when(k % 2 == 1)
        def _odd():
            _step(1)

        @pl.when(k >= 1)
        def _ship():
            for j in range(RT):
                _out_copy(o_ref, s2_ref, sem_out, nb0, k - 1, j).start()

        @pl.when(k == last)
        def _drain():
            for j in range(RT):
                _out_copy(o_ref, s2_ref, sem_out, nb0, last - 2, j).wait()
            for j in range(RT):
                for ns in range(0, NB, 128):
                    piece = band_ref[last % 2, j, :, ns:ns + 128]
                    s2_ref[last % 2, j, pl.ds(ns, 128), :] = piece.T
            for j in range(RT):
                _out_copy(o_ref, s2_ref, sem_out, nb0, last, j).start()
            for kk in (last - 1, last):
                for j in range(RT):
                    _out_copy(o_ref, s2_ref, sem_out, nb0, kk, j).wait()

    return _conv_sig_kernel


def _forward(x_nchw, weight, bias):
    N, C, H, W = x_nchw.shape
    assert C == 1
    Hp = _round_up(PAD + H + PAD, 8)
    Wp = _round_up(WOFF + W + PAD, 8)
    n_bands = H // RT

    a_mat = _banded_a(weight)

    Np = _round_up(N, NB)
    x = x_nchw
    if Np != N:
        x = jnp.pad(x, ((0, Np - N), (0, 0), (0, 0), (0, 0)))

    out = pl.pallas_call(
        _make_kernel(H, W, n_bands),
        out_shape=jax.ShapeDtypeStruct((Np, 1, H, W), x_nchw.dtype),
        grid=(Np // NB, n_bands),
        in_specs=[
            pl.BlockSpec((RT * CT, KDIM), lambda b, h: (0, 0)),
            pl.BlockSpec(memory_space=pltpu.MemorySpace.SMEM),
            pl.BlockSpec(memory_space=pltpu.MemorySpace.HBM),
        ],
        out_specs=pl.BlockSpec(memory_space=pltpu.MemorySpace.HBM),
        scratch_shapes=[
            pltpu.VMEM((Hp, Wp, NB), jnp.bfloat16),
            pltpu.VMEM((H, NB, W), jnp.float32),
            pltpu.VMEM((2, RT, W, NB), jnp.float32),
            pltpu.VMEM((2, RT, NB, W), jnp.float32),
            pltpu.SemaphoreType.DMA((H // HCH, HCH)),
            pltpu.SemaphoreType.DMA((2, RT)),
        ],
        compiler_params=pltpu.CompilerParams(
            dimension_semantics=("parallel", "arbitrary")),
    )(a_mat, bias.astype(jnp.float32), x)

    if Np != N:
        out = out[:N]
    return out


def kernel(x_nchw, weight, bias):
    return _forward(x_nchw, weight, bias)
